# baseline (speedup 1.0000x reference)
import functools

import jax
import jax.numpy as jnp
from jax import lax
from jax.experimental import pallas as pl
from jax.experimental.pallas import tpu as pltpu
from jax.experimental.pallas import tpu_sc as plsc

N_NODES = 10000
EMB = 128
LAYERS = 3
E = 320000

_NC = 2
_NS = 16
_NW = _NC * _NS
_C = 96
_CH = 108
_EWP = _C * _CH
_EP = _EWP * _NW
_RT = N_NODES // _NS
_RO = 1000
_NS_OUT = N_NODES // _RO
_NB = 4


def _spmm_body(ego_hbm, src_hbm, dst_hbm, vals_hbm, zeros_hbm, out_hbm,
               srcs, dsts, vals, rows, acc_sh,
               gsem, lsrc, ldst, lval, ssems):
    cid = lax.axis_index("c")
    sid = lax.axis_index("s")
    wid = sid * _NC + cid
    ebase = wid * _EWP

    pltpu.sync_copy(zeros_hbm, acc_sh.at[pl.ds(sid * _RT, _RT)])
    plsc.subcore_barrier()

    def _load_idx(ci, b, sem_s, sem_v, sem_d, with_dst):
        off = pl.multiple_of(ebase + ci * _C, 8)
        s = pltpu.async_copy(src_hbm.at[pl.ds(off, _C)], srcs[b], sem_s)
        v = pltpu.async_copy(vals_hbm.at[pl.ds(off, _C)],
                             vals[b].at[pl.ds(0, _C)], sem_v)
        d = None
        if with_dst:
            d = pltpu.async_copy(dst_hbm.at[pl.ds(off, _C)], dsts[b], sem_d)
        return s, v, d

    def _gather_start(b_src, b_rows):
        return pltpu.async_copy(ego_hbm.at[srcs[b_src]], rows[b_rows], gsem)

    def _drain_rowsz(sem):
        pltpu.make_async_copy(ego_hbm.at[pl.ds(0, _C)], rows[0], sem).wait()

    def _drain_idx(sem, dst_ref):
        pltpu.make_async_copy(src_hbm.at[pl.ds(0, _C)], dst_ref, sem).wait()

    pltpu.sync_copy(src_hbm.at[pl.ds(pl.multiple_of(ebase, 8), _C)], srcs[0])
    pltpu.sync_copy(dst_hbm.at[pl.ds(pl.multiple_of(ebase, 8), _C)], dsts[0])
    pltpu.sync_copy(vals_hbm.at[pl.ds(pl.multiple_of(ebase, 8), _C)],
                    vals[0].at[pl.ds(0, _C)])
    _gather_start(0, 0)
    _load_idx(1, 1, lsrc, lval, ldst, with_dst=False)

    def _chunk(i, t, p):
        b = p % _NB
        b1 = (p + 1) % _NB
        b2 = (p + 2) % _NB

        @pl.when(i >= 3)
        def _():
            _drain_rowsz(ssems[b1])

        @pl.when(i + 1 < _CH)
        def _():
            off = pl.multiple_of(ebase + (i + 1) * _C, 8)
            pltpu.async_copy(dst_hbm.at[pl.ds(off, _C)], dsts[b1], ldst)

        @pl.when(i < _CH - 1)
        def _():
            _drain_idx(lsrc, srcs[0])
            _drain_idx(lval, vals[0].at[pl.ds(0, _C)])

        @pl.when(i + 2 < _CH)
        def _():
            off = pl.multiple_of(ebase + (i + 2) * _C, 8)
            pltpu.async_copy(src_hbm.at[pl.ds(off, _C)], srcs[b2], lsrc)
            pltpu.async_copy(vals_hbm.at[pl.ds(off, _C)],
                             vals[b2].at[pl.ds(0, _C)], lval)

        _drain_rowsz(gsem)

        @pl.when(i + 1 < _CH)
        def _():
            _gather_start(b1, b1)

        r = rows[b]
        vb = vals[b]

        def edge_body(e, carry):
            v = vb[pl.ds(e, 16)][0]
            for j in range(EMB // 16):
                sl = pl.ds(j * 16, 16)
                r[e, sl] = r[e, sl] * v
            return carry

        lax.fori_loop(0, _C, edge_body, 0, unroll=2)

        @pl.when(i >= 1)
        def _():
            _drain_idx(ldst, dsts[0])

        pltpu.async_copy(r, acc_sh.at[dsts[b]], ssems[b], add=True)

    def loop_body(t, carry):
        for p in range(_NB):
            _chunk(t * _NB + p, t, p)
        return carry

    lax.fori_loop(0, _CH // _NB, loop_body, 0, unroll=False)

    for i in (_CH - 3, _CH - 2, _CH - 1):
        _drain_rowsz(ssems[i % _NB])

    plsc.subcore_barrier()

    @pl.when(sid < _NS_OUT)
    def _copy_out():
        off = pl.multiple_of(sid * _RO, 8)
        pltpu.sync_copy(acc_sh.at[pl.ds(off, _RO)],
                        out_hbm.at[cid, pl.ds(off, _RO)])


_spmm = pl.kernel(
    _spmm_body,
    out_type=jax.ShapeDtypeStruct((_NC, N_NODES, EMB), jnp.float32),
    mesh=plsc.VectorSubcoreMesh(core_axis_name="c", subcore_axis_name="s"),
    scratch_types=[
        [pltpu.VMEM((_C,), jnp.int32) for _ in range(_NB)],
        [pltpu.VMEM((_C,), jnp.int32) for _ in range(_NB)],
        [pltpu.VMEM((_C + 16,), jnp.float32) for _ in range(_NB)],
        [pltpu.VMEM((_C, EMB), jnp.float32) for _ in range(_NB)],
        pltpu.VMEM_SHARED((N_NODES, EMB), jnp.float32),
        pltpu.SemaphoreType.DMA,
        pltpu.SemaphoreType.DMA,
        pltpu.SemaphoreType.DMA,
        pltpu.SemaphoreType.DMA,
        [pltpu.SemaphoreType.DMA for _ in range(_NB)],
    ],
)

_BLK = 1000


def _dense_body(p0_ref, p1_ref, ego_ref, wg_ref, wb_ref, b_ref,
                ego_out_ref, norm_out_ref):
    side = p0_ref[...] + p1_ref[...]
    ego = ego_ref[...]
    x = jnp.dot(side, wg_ref[...], preferred_element_type=jnp.float32)
    x = x + jnp.dot(ego * side, wb_ref[...], preferred_element_type=jnp.float32)
    x = x + b_ref[...]
    y = jnp.where(x > 0, x, 0.2 * x)
    ego_out_ref[...] = y
    nrm = jnp.sqrt(jnp.sum(y * y, axis=1, keepdims=True))
    norm_out_ref[...] = y / jnp.maximum(nrm, 1e-12)


def _dense(p0, p1, ego, wg, wb, b):
    row_spec = pl.BlockSpec((_BLK, EMB), lambda i: (i, 0))
    return pl.pallas_call(
        _dense_body,
        grid=(N_NODES // _BLK,),
        in_specs=[
            row_spec, row_spec, row_spec,
            pl.BlockSpec((EMB, EMB), lambda i: (0, 0)),
            pl.BlockSpec((EMB, EMB), lambda i: (0, 0)),
            pl.BlockSpec((1, EMB), lambda i: (0, 0)),
        ],
        out_specs=[row_spec, row_spec],
        out_shape=[
            jax.ShapeDtypeStruct((N_NODES, EMB), jnp.float32),
            jax.ShapeDtypeStruct((N_NODES, EMB), jnp.float32),
        ],
    )(p0, p1, ego, wg, wb, b)


def kernel(user_emb, item_emb, edge_index, adj_vals, W_gc, b_gc, W_bi, b_bi):
    ego0 = jnp.concatenate([user_emb, item_emb], axis=0)
    pad = _EP - E
    src = jnp.concatenate([edge_index[1], jnp.zeros((pad,), jnp.int32)])
    dst = jnp.concatenate([edge_index[0], jnp.zeros((pad,), jnp.int32)])
    vals = jnp.concatenate([adj_vals, jnp.zeros((pad,), jnp.float32)])
    zeros = jnp.zeros((_RT, EMB), jnp.float32)
    b_tot = b_gc + b_bi
    outs = [ego0]
    ego = ego0
    for k in range(LAYERS):
        parts = _spmm(ego, src, dst, vals, zeros)
        ego, norm = _dense(parts[0], parts[1], ego, W_gc[k], W_bi[k], b_tot[k])
        outs.append(norm)
    return jnp.concatenate(outs, axis=1)


# speedup vs baseline: 1.8076x; 1.8076x over previous
import functools

import jax
import jax.numpy as jnp
from jax import lax
from jax.experimental import pallas as pl
from jax.experimental.pallas import tpu as pltpu
from jax.experimental.pallas import tpu_sc as plsc

N_NODES = 10000
EMB = 128
LAYERS = 3
E = 320000

_NC = 2
_NS = 16
_NW = _NC * _NS
_C = 80
_CH = 125
_EW = _C * _CH
_RT = N_NODES // _NS
_RO = 1000
_NS_OUT = N_NODES // _RO
_NB = 2


def _spmm_body(ego_hbm, src_hbm, dst_hbm, vals_hbm, zeros_hbm, out_hbm,
               srcs, dsts, vals, rows, acc_sh, gsem):
    cid = lax.axis_index("c")
    sid = lax.axis_index("s")
    wid = sid * _NC + cid
    ebase = wid * _EW

    pltpu.sync_copy(zeros_hbm, acc_sh.at[pl.ds(sid * _RT, _RT)])
    plsc.subcore_barrier()

    def _load_idx(ci, b):
        off = pl.multiple_of(ebase + ci * _C, 8)
        pltpu.sync_copy(src_hbm.at[pl.ds(off, _C)], srcs[b])
        pltpu.sync_copy(dst_hbm.at[pl.ds(off, _C)], dsts[b])
        pltpu.sync_copy(vals_hbm.at[pl.ds(off, _C)], vals[b].at[pl.ds(0, _C)])

    def _gather_start(b):
        return pltpu.async_copy(ego_hbm.at[srcs[b]], rows[b], gsem)

    def _drain_gather():
        pltpu.make_async_copy(ego_hbm.at[pl.ds(0, _C)], rows[0], gsem).wait()

    _load_idx(0, 0)
    _gather_start(0)

    def _chunk(i, p, last):
        b = p % _NB
        nb = (p + 1) % _NB
        if not last:
            _load_idx(i + 1, nb)
        _drain_gather()
        if not last:
            _gather_start(nb)

        r = rows[b]
        vb = vals[b]

        def edge_body(e, carry):
            v = vb[pl.ds(e, 16)][0]
            for j in range(EMB // 16):
                sl = pl.ds(j * 16, 16)
                r[e, sl] = r[e, sl] * v
            return carry

        lax.fori_loop(0, _C, edge_body, 0, unroll=2)
        pltpu.sync_copy(r, acc_sh.at[dsts[b]], add=True)

    def loop_body(t, carry):
        for p in range(_NB):
            _chunk(t * _NB + p, p, last=False)
        return carry

    lax.fori_loop(0, (_CH - 1) // _NB, loop_body, 0, unroll=False)
    _chunk(_CH - 1, (_CH - 1) % _NB, last=True)

    plsc.subcore_barrier()

    @pl.when(sid < _NS_OUT)
    def _copy_out():
        off = pl.multiple_of(sid * _RO, 8)
        pltpu.sync_copy(acc_sh.at[pl.ds(off, _RO)],
                        out_hbm.at[cid, pl.ds(off, _RO)])


_spmm = pl.kernel(
    _spmm_body,
    out_type=jax.ShapeDtypeStruct((_NC, N_NODES, EMB), jnp.float32),
    mesh=plsc.VectorSubcoreMesh(core_axis_name="c", subcore_axis_name="s"),
    scratch_types=[
        [pltpu.VMEM((_C,), jnp.int32) for _ in range(_NB)],
        [pltpu.VMEM((_C,), jnp.int32) for _ in range(_NB)],
        [pltpu.VMEM((_C + 16,), jnp.float32) for _ in range(_NB)],
        [pltpu.VMEM((_C, EMB), jnp.float32) for _ in range(_NB)],
        pltpu.VMEM_SHARED((N_NODES, EMB), jnp.float32),
        pltpu.SemaphoreType.DMA,
    ],
)

_BLK = 1000


def _dense_body(p0_ref, p1_ref, ego_ref, wg_ref, wb_ref, b_ref,
                ego_out_ref, norm_out_ref):
    side = p0_ref[...] + p1_ref[...]
    ego = ego_ref[...]
    x = jnp.dot(side, wg_ref[...], preferred_element_type=jnp.float32)
    x = x + jnp.dot(ego * side, wb_ref[...], preferred_element_type=jnp.float32)
    x = x + b_ref[...]
    y = jnp.where(x > 0, x, 0.2 * x)
    ego_out_ref[...] = y
    nrm = jnp.sqrt(jnp.sum(y * y, axis=1, keepdims=True))
    norm_out_ref[...] = y / jnp.maximum(nrm, 1e-12)


def _dense(p0, p1, ego, wg, wb, b):
    row_spec = pl.BlockSpec((_BLK, EMB), lambda i: (i, 0))
    return pl.pallas_call(
        _dense_body,
        grid=(N_NODES // _BLK,),
        in_specs=[
            row_spec, row_spec, row_spec,
            pl.BlockSpec((EMB, EMB), lambda i: (0, 0)),
            pl.BlockSpec((EMB, EMB), lambda i: (0, 0)),
            pl.BlockSpec((1, EMB), lambda i: (0, 0)),
        ],
        out_specs=[row_spec, row_spec],
        out_shape=[
            jax.ShapeDtypeStruct((N_NODES, EMB), jnp.float32),
            jax.ShapeDtypeStruct((N_NODES, EMB), jnp.float32),
        ],
    )(p0, p1, ego, wg, wb, b)


def kernel(user_emb, item_emb, edge_index, adj_vals, W_gc, b_gc, W_bi, b_bi):
    ego0 = jnp.concatenate([user_emb, item_emb], axis=0)
    src = edge_index[1]
    dst = edge_index[0]
    vals = adj_vals
    zeros = jnp.zeros((_RT, EMB), jnp.float32)
    b_tot = b_gc + b_bi
    outs = [ego0]
    ego = ego0
    for k in range(LAYERS):
        parts = _spmm(ego, src, dst, vals, zeros)
        ego, norm = _dense(parts[0], parts[1], ego, W_gc[k], W_bi[k], b_tot[k])
        outs.append(norm)
    return jnp.concatenate(outs, axis=1)
